# baseline (speedup 1.0000x reference)
import jax
import jax.numpy as jnp
from jax.experimental import pallas as pl
from jax.experimental.pallas import tpu as pltpu


NT = 512
GRID = 4096 // NT


def _encode(z):
    v = jax.lax.bitcast_convert_type(z, jnp.int32)
    s = (v >> 16) & 0x8000
    a = (v & 0x7FFFFFFF) + 0x1000
    u = jnp.maximum(a - 0x38000000, 0)
    return s | (u >> 13)


def _body(yl_ref, ys_ref, wl_ref, ws_ref, bl_ref, bs_ref, out_ref,
          yscl_scr, yscs_scr):
    @pl.when(pl.program_id(0) == 0)
    def _init():
        iota = jax.lax.broadcasted_iota(jnp.int32, (128, 1024), 1)
        zero = jnp.bfloat16(0)
        yl = yl_ref[...].astype(jnp.bfloat16)
        t_l = jnp.concatenate([yl] * 16, axis=1)
        yscl_scr[...] = jnp.where((iota >> 6) == wl_ref[...], t_l, zero)
        ys = ys_ref[...].astype(jnp.bfloat16)
        t_s = jnp.concatenate([ys] * 64, axis=1)
        yscs_scr[...] = jnp.where((iota >> 4) == ws_ref[...], t_s, zero)

    dn = (((1,), (0,)), ((), ()))
    zl = jax.lax.dot_general(yscl_scr[...], bl_ref[...], dn,
                             preferred_element_type=jnp.float32) * 2.0
    zs = jax.lax.dot_general(yscs_scr[...], bs_ref[...], dn,
                             preferred_element_type=jnp.float32) * 2.0

    ob = out_ref.bitcast(jnp.int32)
    ob[0:64, :] = _encode(zl[0:64]) | (_encode(zl[64:128]) << 16)
    ob[64:128, :] = _encode(zs[0:64]) | (_encode(zs[64:128]) << 16)


@jax.jit
def kernel(y_large, y_small, wids_large, wids_small, lora_B_large, lora_B_small):
    perm = jnp.concatenate([jnp.arange(0, 128, 2, dtype=jnp.int32),
                            jnp.arange(1, 128, 2, dtype=jnp.int32)])
    ylp = y_large.reshape(128, 64)[perm].astype(jnp.float32)
    ysp = y_small.reshape(128, 16)[perm].astype(jnp.float32)
    wl = wids_large[perm].reshape(128, 1)
    ws = wids_small[perm].reshape(128, 1)
    bl = lora_B_large.reshape(16 * 64, 4096).astype(jnp.bfloat16)
    bs = lora_B_small.reshape(64 * 16, 4096).astype(jnp.bfloat16)

    out = pl.pallas_call(
        _body,
        grid=(GRID,),
        in_specs=[
            pl.BlockSpec((128, 64), lambda n: (0, 0)),
            pl.BlockSpec((128, 16), lambda n: (0, 0)),
            pl.BlockSpec((128, 1), lambda n: (0, 0)),
            pl.BlockSpec((128, 1), lambda n: (0, 0)),
            pl.BlockSpec((1024, NT), lambda n: (0, n)),
            pl.BlockSpec((1024, NT), lambda n: (0, n)),
        ],
        out_specs=pl.BlockSpec((256, NT), lambda n: (0, n)),
        out_shape=jax.ShapeDtypeStruct((256, 4096), jnp.bfloat16),
        scratch_shapes=[
            pltpu.VMEM((128, 1024), jnp.bfloat16),
            pltpu.VMEM((128, 1024), jnp.bfloat16),
        ],
    )(ylp, ysp, wl, ws, bl, bs)
    z = jax.lax.bitcast_convert_type(out, jnp.float16)
    return z.reshape(256, 1, 4096)
